# Initial kernel scaffold; baseline (speedup 1.0000x reference)
import jax
import jax.numpy as jnp
from jax.experimental import pallas as pl


def kernel(features, batch_idx, yx_idx, Wk, gamma, beta, running_mean, running_var):
    raise NotImplementedError("write your pallas kernel here")


# speedup vs baseline: 1.0163x; 1.0163x over previous
import jax
import jax.numpy as jnp
from jax.experimental import pallas as pl

_B, _H, _CIN, _COUT = 4, 512, 32, 64
_HO = _H // 2
_EPS = 1e-5
_R = 8
_OCC_C = 8

_TERMS = ((0, 0, 1), (1, 0, 2), (1, 1, 0))


def _conv_bn_relu_kernel(pc_ref, pp_ref, oc_ref, op_ref, w_ref,
                         scale_ref, bias_ref, out_ref):
    r = pl.program_id(1)
    halo_on = jnp.where(r == 0, 0.0, 1.0)
    acc = jnp.zeros((_R * _HO, _COUT), dtype=jnp.float32)
    occ = jnp.zeros((_R, _HO, _OCC_C), dtype=jnp.float32)
    for py, sy, wy in _TERMS:
        for px, sx, wx in _TERMS:
            if sy:
                prev = pp_ref[0, 0, px] * halo_on
                x = jnp.concatenate([prev, pc_ref[0, 1, px][:-1]], axis=0)
                oprev = op_ref[0, 0, px] * halo_on
                oc = jnp.concatenate([oprev, oc_ref[0, 1, px][:-1]], axis=0)
            else:
                x = pc_ref[0, py, px]
                oc = oc_ref[0, py, px]
            if sx:
                x = jnp.concatenate(
                    [jnp.zeros((_R, 1, _CIN), jnp.float32), x[:, :-1]], axis=1)
                oc = jnp.concatenate(
                    [jnp.zeros((_R, 1, _OCC_C), jnp.float32), oc[:, :-1]], axis=1)
            acc = acc + jnp.dot(
                x.reshape(_R * _HO, _CIN), w_ref[wy, wx],
                preferred_element_type=jnp.float32)
            occ = occ + oc
    y = acc * scale_ref[0] + bias_ref[0]
    y = jnp.maximum(y, 0.0)
    occ2 = occ.reshape(_R * _HO, _OCC_C)
    y = jnp.where(occ2[:, 0:1] > 0.0, y, 0.0)
    out_ref[0] = y.reshape(_R, _HO, _COUT)


def kernel(features, batch_idx, yx_idx, Wk, gamma, beta, running_mean, running_var):
    y_i = yx_idx[:, 0]
    x_i = yx_idx[:, 1]
    py = y_i & 1
    px = x_i & 1
    hy = y_i >> 1
    hx = x_i >> 1

    P = jnp.zeros((_B, 2, 2, _HO, _HO, _CIN), jnp.float32)
    P = P.at[batch_idx, py, px, hy, hx].add(features)
    O = jnp.zeros((_B, 2, 2, _HO, _HO, _OCC_C), jnp.float32)
    O = O.at[batch_idx, py, px, hy, hx, 0].add(1.0)

    scale = gamma / jnp.sqrt(running_var + _EPS)
    bias = beta - running_mean * scale

    return pl.pallas_call(
        _conv_bn_relu_kernel,
        grid=(_B, _HO // _R),
        in_specs=[
            pl.BlockSpec((1, 2, 2, _R, _HO, _CIN),
                         lambda b, r: (b, 0, 0, r, 0, 0)),
            pl.BlockSpec((1, 1, 2, 1, _HO, _CIN),
                         lambda b, r: (b, 1, 0, jnp.maximum(r * _R - 1, 0), 0, 0)),
            pl.BlockSpec((1, 2, 2, _R, _HO, _OCC_C),
                         lambda b, r: (b, 0, 0, r, 0, 0)),
            pl.BlockSpec((1, 1, 2, 1, _HO, _OCC_C),
                         lambda b, r: (b, 1, 0, jnp.maximum(r * _R - 1, 0), 0, 0)),
            pl.BlockSpec((3, 3, _CIN, _COUT), lambda b, r: (0, 0, 0, 0)),
            pl.BlockSpec((1, _COUT), lambda b, r: (0, 0)),
            pl.BlockSpec((1, _COUT), lambda b, r: (0, 0)),
        ],
        out_specs=pl.BlockSpec((1, _R, _HO, _COUT), lambda b, r: (b, r, 0, 0)),
        out_shape=jax.ShapeDtypeStruct((_B, _HO, _HO, _COUT), jnp.float32),
    )(P, P, O, O, Wk, scale.reshape(1, _COUT), bias.reshape(1, _COUT))


# speedup vs baseline: 1.1011x; 1.0834x over previous
import functools

import jax
import jax.numpy as jnp
from jax import lax
from jax.experimental import pallas as pl
from jax.experimental.pallas import tpu as pltpu
from jax.experimental.pallas import tpu_sc as plsc

_B, _H, _CIN, _COUT = 4, 512, 32, 64
_HO = _H // 2
_EPS = 1e-5

_N = 200000
_NSUB = 16
_NPAD = 200704
_CHUNK = _NPAD // _NSUB
_NCH = _CIN + 1
_CELLS = _B * 2 * 2 * _HO * _HO
_SPCELLS = _CELLS + 16
_PER_TEC = _CELLS // 16
_ZCHUNK = 8192
_ROUNDS = (_NCH + 1) // 2

_R = 8
_OCC_C = 8

_TERMS = ((0, 0, 1), (1, 0, 2), (1, 1, 0))


def _sc_scatter(b_hbm, y_hbm, x_hbm, vals_hbm, out_hbm,
                st_v, idx_v, vals_v, zbuf, plane):
    c = lax.axis_index("c")
    s = lax.axis_index("s")
    base = s * _CHUNK

    def _accum(scale_even, scale_odd1, scale_odd2):
        def body(j, carry):
            i = j // 8
            k = j % 8
            v = st_v[pl.ds(j * 16, 16)]
            contrib = (scale_even * v if scale_odd1 is None else
                       jnp.bitwise_and(v, 1) * scale_odd1
                       + jnp.right_shift(v, 1) * scale_odd2)
            prev = idx_v[i, pl.ds(k * 16, 16)]
            idx_v[i, pl.ds(k * 16, 16)] = prev + contrib
            return carry
        return body

    def _init(j, carry):
        i = j // 8
        k = j % 8
        idx_v[i, pl.ds(k * 16, 16)] = st_v[pl.ds(j * 16, 16)] * 262144
        return carry

    pltpu.sync_copy(b_hbm.at[pl.ds(base, _CHUNK)], st_v)
    lax.fori_loop(0, _CHUNK // 16, _init, 0)
    pltpu.sync_copy(y_hbm.at[pl.ds(base, _CHUNK)], st_v)
    lax.fori_loop(0, _CHUNK // 16, _accum(None, 131072, 256), 0)
    pltpu.sync_copy(x_hbm.at[pl.ds(base, _CHUNK)], st_v)
    lax.fori_loop(0, _CHUNK // 16, _accum(None, 65536, 1), 0)

    def _zero16(j, carry):
        zbuf[pl.ds(j * 16, 16)] = jnp.zeros((16,), jnp.float32)
        return carry

    lax.fori_loop(0, _ZCHUNK // 16, _zero16, 0)

    for r in range(_ROUNDS):
        ch = 2 * r + c

        @pl.when(ch < _NCH)
        def _round():
            for k in range(_PER_TEC // _ZCHUNK):
                pltpu.sync_copy(
                    zbuf, plane.at[pl.ds(s * _PER_TEC + k * _ZCHUNK, _ZCHUNK)])

            @pl.when(s == 0)
            def _zero_dump():
                pltpu.sync_copy(zbuf.at[pl.ds(0, 16)], plane.at[pl.ds(_CELLS, 16)])

            plsc.subcore_barrier()

            pltpu.sync_copy(vals_hbm.at[ch, s], vals_v)
            def _scat(j, carry):
                pltpu.sync_copy(vals_v.at[j], plane.at[idx_v.at[j]], add=True)
                return carry

            lax.fori_loop(0, _CHUNK // 128, _scat, 0)
            plsc.subcore_barrier()

            pltpu.sync_copy(
                plane.at[pl.ds(s * _PER_TEC, _PER_TEC)],
                out_hbm.at[pl.ds(ch * _CELLS + s * _PER_TEC, _PER_TEC)])
            plsc.subcore_barrier()


def _scatter_phase_grids(features, batch_idx, yx_idx):
    pad = _NPAD - _N
    b_pad = jnp.concatenate(
        [batch_idx, jnp.full((pad,), _B, jnp.int32)])
    y_pad = jnp.concatenate([yx_idx[:, 0], jnp.zeros((pad,), jnp.int32)])
    x_pad = jnp.concatenate([yx_idx[:, 1], jnp.zeros((pad,), jnp.int32)])
    occ_col = jnp.ones((_N, 1), jnp.float32)
    vals = jnp.concatenate([features, occ_col], axis=1).T
    vals = jnp.concatenate(
        [vals, jnp.zeros((_NCH, pad), jnp.float32)],
        axis=1).reshape(_NCH, _NSUB, _CHUNK // 128, 128)

    mesh = plsc.VectorSubcoreMesh(core_axis_name="c", subcore_axis_name="s")
    run = functools.partial(
        pl.kernel, mesh=mesh,
        out_type=jax.ShapeDtypeStruct((_NCH * _CELLS,), jnp.float32),
        scratch_types=[
            pltpu.VMEM((_CHUNK,), jnp.int32),
            pltpu.VMEM((_CHUNK // 128, 128), jnp.int32),
            pltpu.VMEM((_CHUNK // 128, 128), jnp.float32),
            pltpu.VMEM((_ZCHUNK,), jnp.float32),
            pltpu.VMEM_SHARED((_SPCELLS,), jnp.float32),
        ],
    )(_sc_scatter)
    return run(b_pad, y_pad, x_pad, vals)


def _conv_bn_relu_kernel(pc_ref, pp_ref, oc_ref, op_ref, w_ref,
                         scale_ref, bias_ref, out_ref):
    r = pl.program_id(1)
    halo_on = jnp.where(r == 0, 0.0, 1.0)
    acc = jnp.zeros((_R * _HO, _COUT), dtype=jnp.float32)
    occ = jnp.zeros((_R, _HO, _OCC_C), dtype=jnp.float32)
    for py, sy, wy in _TERMS:
        for px, sx, wx in _TERMS:
            if sy:
                prev = pp_ref[0, 0, px] * halo_on
                x = jnp.concatenate([prev, pc_ref[0, 1, px][:-1]], axis=0)
                oprev = op_ref[0, 0, px] * halo_on
                oc = jnp.concatenate([oprev, oc_ref[0, 1, px][:-1]], axis=0)
            else:
                x = pc_ref[0, py, px]
                oc = oc_ref[0, py, px]
            if sx:
                x = jnp.concatenate(
                    [jnp.zeros((_R, 1, _CIN), jnp.float32), x[:, :-1]], axis=1)
                oc = jnp.concatenate(
                    [jnp.zeros((_R, 1, _OCC_C), jnp.float32), oc[:, :-1]], axis=1)
            acc = acc + jnp.dot(
                x.reshape(_R * _HO, _CIN), w_ref[wy, wx],
                preferred_element_type=jnp.float32)
            occ = occ + oc
    y = acc * scale_ref[0] + bias_ref[0]
    y = jnp.maximum(y, 0.0)
    occ2 = occ.reshape(_R * _HO, _OCC_C)
    y = jnp.where(occ2[:, 0:1] > 0.0, y, 0.0)
    out_ref[0] = y.reshape(_R, _HO, _COUT)


def kernel(features, batch_idx, yx_idx, Wk, gamma, beta, running_mean, running_var):
    grids = _scatter_phase_grids(features, batch_idx, yx_idx)
    grids = grids.reshape(_NCH, _B, 2, 2, _HO, _HO)
    P = jnp.transpose(grids[:_CIN], (1, 2, 3, 4, 5, 0))
    occ_plane = grids[_CIN]
    O = jnp.zeros((_B, 2, 2, _HO, _HO, _OCC_C), jnp.float32)
    O = O.at[..., 0].set(occ_plane)

    scale = gamma / jnp.sqrt(running_var + _EPS)
    bias = beta - running_mean * scale

    return pl.pallas_call(
        _conv_bn_relu_kernel,
        grid=(_B, _HO // _R),
        in_specs=[
            pl.BlockSpec((1, 2, 2, _R, _HO, _CIN),
                         lambda b, r: (b, 0, 0, r, 0, 0)),
            pl.BlockSpec((1, 1, 2, 1, _HO, _CIN),
                         lambda b, r: (b, 1, 0, jnp.maximum(r * _R - 1, 0), 0, 0)),
            pl.BlockSpec((1, 2, 2, _R, _HO, _OCC_C),
                         lambda b, r: (b, 0, 0, r, 0, 0)),
            pl.BlockSpec((1, 1, 2, 1, _HO, _OCC_C),
                         lambda b, r: (b, 1, 0, jnp.maximum(r * _R - 1, 0), 0, 0)),
            pl.BlockSpec((3, 3, _CIN, _COUT), lambda b, r: (0, 0, 0, 0)),
            pl.BlockSpec((1, _COUT), lambda b, r: (0, 0)),
            pl.BlockSpec((1, _COUT), lambda b, r: (0, 0)),
        ],
        out_specs=pl.BlockSpec((1, _R, _HO, _COUT), lambda b, r: (b, r, 0, 0)),
        out_shape=jax.ShapeDtypeStruct((_B, _HO, _HO, _COUT), jnp.float32),
    )(P, P, O, O, Wk, scale.reshape(1, _COUT), bias.reshape(1, _COUT))
